# baseline (speedup 1.0000x reference)
import jax
import jax.numpy as jnp
from jax.experimental import pallas as pl
from jax.experimental.pallas import tpu as pltpu

B = 8
S = 1024
D = 768
P = 4096

P_BLK = 2048
N_PT = P // P_BLK
N_MM = N_PT * B

_PREC = jax.lax.Precision.DEFAULT


def _proto_kernel(x_ref, p_ref, dist_ref, idx_ref, pn_ref, dots_ref):
    g = pl.program_id(0)

    @pl.when(g % B == 0)
    def _():
        pblk = p_ref[...]
        ss = jnp.sum(pblk * pblk, axis=1, keepdims=True)
        pn_ref[...] = pblk * jax.lax.rsqrt(jnp.maximum(ss, 1e-24))

    xblk = x_ref[0]
    ssx = jnp.sum(xblk * xblk, axis=1, keepdims=True)
    xn = xblk * jax.lax.rsqrt(jnp.maximum(ssx, 1e-24))

    def _step(wr_slot, rd_slot):
        dots_ref[wr_slot] = jax.lax.dot_general(
            xn, pn_ref[...],
            dimension_numbers=(((1,), (1,)), ((), ())),
            precision=_PREC,
            preferred_element_type=jnp.float32,
        )
        d = dots_ref[rd_slot]
        cmax = jnp.max(d, axis=0, keepdims=True)
        cidx = jnp.argmax(d, axis=0, keepdims=True).astype(jnp.int32)
        dist_ref[0] = 1.0 - cmax
        idx_ref[0] = cidx

    pl.when(g % 2 == 0)(lambda: _step(0, 1))
    pl.when(g % 2 == 1)(lambda: _step(1, 0))


@jax.jit
def kernel(x, prototypes):
    grid = (N_MM + 1,)

    def x_map(g):
        gc = jnp.minimum(g, N_MM - 1)
        return (gc % B, 0, 0)

    def p_map(g):
        gc = jnp.minimum(g, N_MM - 1)
        return (gc // B, 0)

    def out_map(g):
        gm = jnp.maximum(g, 1) - 1
        return (gm % B, 0, gm // B)

    dist, idx = pl.pallas_call(
        _proto_kernel,
        grid=grid,
        in_specs=[
            pl.BlockSpec((1, S, D), x_map),
            pl.BlockSpec((P_BLK, D), p_map),
        ],
        out_specs=[
            pl.BlockSpec((1, 1, P_BLK), out_map),
            pl.BlockSpec((1, 1, P_BLK), out_map),
        ],
        out_shape=[
            jax.ShapeDtypeStruct((B, 1, P), jnp.float32),
            jax.ShapeDtypeStruct((B, 1, P), jnp.int32),
        ],
        scratch_shapes=[
            pltpu.VMEM((P_BLK, D), jnp.float32),
            pltpu.VMEM((2, S, P_BLK), jnp.float32),
        ],
    )(x, prototypes)
    return dist, idx.astype(jnp.int64)


# speedup vs baseline: 1.4469x; 1.4469x over previous
import jax
import jax.numpy as jnp
from jax.experimental import pallas as pl
from jax.experimental.pallas import tpu as pltpu

B = 8
S = 1024
D = 768
P = 4096

P_BLK = 2048
S_CHUNK = 256
N_PT = P // P_BLK
N_CHUNK = S // S_CHUNK
R_CHUNK = S_CHUNK // 8
P_SUB = P_BLK

_PREC = jax.lax.Precision.DEFAULT


def _proto_kernel(x_ref, p_ref, dist_ref, idx_ref, pn_ref):
    b = pl.program_id(1)

    @pl.when(b == 0)
    def _():
        pblk = p_ref[...]
        ss = jnp.sum(pblk * pblk, axis=1, keepdims=True)
        pn_ref[...] = pblk * jax.lax.rsqrt(jnp.maximum(ss, 1e-24))

    pn = pn_ref[...]

    m8 = None
    mi8 = None
    for c in range(N_CHUNK):
        xc = x_ref[0, c * S_CHUNK:(c + 1) * S_CHUNK, :]
        ssx = jnp.sum(xc * xc, axis=1, keepdims=True)
        xn = xc * jax.lax.rsqrt(jnp.maximum(ssx, 1e-24))
        dots = jax.lax.dot_general(
            xn, pn,
            dimension_numbers=(((1,), (1,)), ((), ())),
            precision=_PREC,
            preferred_element_type=jnp.float32,
        )
        dr = dots.reshape(R_CHUNK, 8, P_BLK)
        for i in range(R_CHUNK):
            di = dr[i]
            gi = c * R_CHUNK + i
            if m8 is None:
                m8 = di
                mi8 = jnp.zeros((8, P_BLK), jnp.int32)
            else:
                mask = di > m8
                m8 = jnp.where(mask, di, m8)
                mi8 = jnp.where(mask, gi, mi8)

    row8 = mi8 * 8 + jax.lax.broadcasted_iota(jnp.int32, (8, P_BLK), 0)
    for sh in (4, 2, 1):
        m2 = pltpu.roll(m8, sh, axis=0)
        r2 = pltpu.roll(row8, sh, axis=0)
        better = (m2 > m8) | ((m2 == m8) & (r2 < row8))
        m8 = jnp.where(better, m2, m8)
        row8 = jnp.where(better, r2, row8)

    dist_ref[0] = 1.0 - m8[0:1]
    idx_ref[0] = row8[0:1]


@jax.jit
def kernel(x, prototypes):
    grid = (N_PT, B)
    dist, idx = pl.pallas_call(
        _proto_kernel,
        grid=grid,
        in_specs=[
            pl.BlockSpec((1, S, D), lambda pt, b: (b, 0, 0)),
            pl.BlockSpec((P_BLK, D), lambda pt, b: (pt, 0)),
        ],
        out_specs=[
            pl.BlockSpec((1, 1, P_BLK), lambda pt, b: (b, 0, pt)),
            pl.BlockSpec((1, 1, P_BLK), lambda pt, b: (b, 0, pt)),
        ],
        out_shape=[
            jax.ShapeDtypeStruct((B, 1, P), jnp.float32),
            jax.ShapeDtypeStruct((B, 1, P), jnp.int32),
        ],
        scratch_shapes=[pltpu.VMEM((P_BLK, D), jnp.float32)],
    )(x, prototypes)
    return dist, idx.astype(jnp.int64)
